# Initial kernel scaffold; baseline (speedup 1.0000x reference)
import jax
import jax.numpy as jnp
from jax.experimental import pallas as pl


def kernel(stimulus_set, embedding):
    raise NotImplementedError("write your pallas kernel here")


# speedup vs baseline: 5.5298x; 5.5298x over previous
import jax
import jax.numpy as jnp
from jax import lax
from jax.experimental import pallas as pl
from jax.experimental.pallas import tpu as pltpu
from jax.experimental.pallas import tpu_sc as plsc

N_STIMULI = 20
N_DIM = 3
BETA = 10.0
GAMMA = 0.001
BATCH = 16384

NUM_CORES = 2
NUM_SUBCORES = 16
LANES = 16
NUM_WORKERS = NUM_CORES * NUM_SUBCORES
B_PER_W = BATCH // NUM_WORKERS
STEPS = B_PER_W // LANES
TABLE_WORDS = (N_STIMULI + 1) * N_DIM


def _sqrt16(x):
    i = plsc.bitcast(x, jnp.int32)
    i = jnp.int32(0x5F3759DF) - lax.shift_right_arithmetic(i, 1)
    y = plsc.bitcast(i, jnp.float32)
    xh = x * jnp.float32(0.5)
    for _ in range(3):
        y = y * (jnp.float32(1.5) - xh * y * y)
    return x * y


def _sc_body(stim_hbm, emb_hbm, out_hbm, idx_v, emb_v, out_v):
    wid = lax.axis_index("s") * NUM_CORES + lax.axis_index("c")
    base = wid * B_PER_W

    pltpu.sync_copy(stim_hbm.at[pl.ds(base * N_DIM, B_PER_W * N_DIM)], idx_v)
    pltpu.sync_copy(emb_hbm, emb_v)

    lanes3 = lax.iota(jnp.int32, LANES) * jnp.int32(N_DIM)
    one = jnp.full((LANES,), 1, jnp.int32)

    for step in range(STEPS):
        e3 = lanes3 + jnp.int32(step * LANES * N_DIM)
        q3 = plsc.load_gather(idx_v, [e3]) * jnp.int32(N_DIM)
        r13 = plsc.load_gather(idx_v, [e3 + one]) * jnp.int32(N_DIM)
        r23 = plsc.load_gather(idx_v, [e3 + one + one]) * jnp.int32(N_DIM)

        dsq1 = jnp.full((LANES,), 0.0, jnp.float32)
        dsq2 = jnp.full((LANES,), 0.0, jnp.float32)
        for d in range(N_DIM):
            dd = jnp.full((LANES,), d, jnp.int32)
            zq = plsc.load_gather(emb_v, [q3 + dd])
            zr1 = plsc.load_gather(emb_v, [r13 + dd])
            zr2 = plsc.load_gather(emb_v, [r23 + dd])
            t1 = zq - zr1
            t2 = zq - zr2
            dsq1 = dsq1 + t1 * t1
            dsq2 = dsq2 + t2 * t2

        s1 = jnp.exp(jnp.float32(-BETA) * _sqrt16(dsq1)) + jnp.float32(GAMMA)
        s2 = jnp.exp(jnp.float32(-BETA) * _sqrt16(dsq2)) + jnp.float32(GAMMA)
        inv = jnp.float32(1.0) / (s1 + s2)

        o1 = lanes3 - lax.iota(jnp.int32, LANES) + jnp.int32(step * LANES * 2)
        plsc.store_scatter(out_v, [o1], s1 * inv)
        plsc.store_scatter(out_v, [o1 + one], s2 * inv)

    pltpu.sync_copy(out_v, out_hbm.at[pl.ds(base * 2, 2 * B_PER_W)])


@jax.jit
def kernel(stimulus_set, embedding):
    mesh = plsc.VectorSubcoreMesh(
        core_axis_name="c", subcore_axis_name="s",
        num_cores=NUM_CORES, num_subcores=NUM_SUBCORES,
    )
    out = pl.kernel(
        _sc_body,
        out_type=jax.ShapeDtypeStruct((2 * BATCH,), jnp.float32),
        mesh=mesh,
        compiler_params=pltpu.CompilerParams(needs_layout_passes=False),
        scratch_types=[
            pltpu.VMEM((B_PER_W * N_DIM,), jnp.int32),
            pltpu.VMEM((TABLE_WORDS,), jnp.float32),
            pltpu.VMEM((2 * B_PER_W,), jnp.float32),
        ],
    )(stimulus_set.reshape(-1), embedding.reshape(-1))
    return out.reshape(BATCH, 2)


# speedup vs baseline: 10.2062x; 1.8457x over previous
import jax
import jax.numpy as jnp
from jax import lax
from jax.experimental import pallas as pl
from jax.experimental.pallas import tpu as pltpu
from jax.experimental.pallas import tpu_sc as plsc

N_STIMULI = 20
N_DIM = 3
BETA = 10.0
GAMMA = 0.001
BATCH = 16384

NUM_CORES = 2
NUM_SUBCORES = 16
LANES = 16
NUM_WORKERS = NUM_CORES * NUM_SUBCORES
B_PER_W = BATCH // NUM_WORKERS
STEPS = B_PER_W // LANES
TABLE_WORDS = (N_STIMULI + 1) * N_DIM


def _sqrt16(x):
    i = plsc.bitcast(x, jnp.int32)
    i = jnp.int32(0x5F3759DF) - lax.shift_right_arithmetic(i, 1)
    y = plsc.bitcast(i, jnp.float32)
    xh = x * jnp.float32(0.5)
    for _ in range(3):
        y = y * (jnp.float32(1.5) - xh * y * y)
    return x * y


def _sc_body(stim_hbm, emb_hbm, out_hbm, q_v, r1_v, r2_v, emb_v, p1_v, p2_v):
    wid = lax.axis_index("s") * NUM_CORES + lax.axis_index("c")
    base = wid * B_PER_W

    pltpu.sync_copy(stim_hbm.at[0, pl.ds(base, B_PER_W)], q_v)
    pltpu.sync_copy(stim_hbm.at[1, pl.ds(base, B_PER_W)], r1_v)
    pltpu.sync_copy(stim_hbm.at[2, pl.ds(base, B_PER_W)], r2_v)
    pltpu.sync_copy(emb_hbm, emb_v)

    for step in range(STEPS):
        off = step * LANES
        q3 = q_v[pl.ds(off, LANES)] * jnp.int32(N_DIM)
        r13 = r1_v[pl.ds(off, LANES)] * jnp.int32(N_DIM)
        r23 = r2_v[pl.ds(off, LANES)] * jnp.int32(N_DIM)

        dsq1 = jnp.full((LANES,), 0.0, jnp.float32)
        dsq2 = jnp.full((LANES,), 0.0, jnp.float32)
        for d in range(N_DIM):
            dd = jnp.full((LANES,), d, jnp.int32)
            zq = plsc.load_gather(emb_v, [q3 + dd])
            zr1 = plsc.load_gather(emb_v, [r13 + dd])
            zr2 = plsc.load_gather(emb_v, [r23 + dd])
            t1 = zq - zr1
            t2 = zq - zr2
            dsq1 = dsq1 + t1 * t1
            dsq2 = dsq2 + t2 * t2

        s1 = jnp.exp(jnp.float32(-BETA) * _sqrt16(dsq1)) + jnp.float32(GAMMA)
        s2 = jnp.exp(jnp.float32(-BETA) * _sqrt16(dsq2)) + jnp.float32(GAMMA)
        inv = jnp.float32(1.0) / (s1 + s2)

        p1_v[pl.ds(off, LANES)] = s1 * inv
        p2_v[pl.ds(off, LANES)] = s2 * inv

    pltpu.sync_copy(p1_v, out_hbm.at[0, pl.ds(base, B_PER_W)])
    pltpu.sync_copy(p2_v, out_hbm.at[1, pl.ds(base, B_PER_W)])


@jax.jit
def kernel(stimulus_set, embedding):
    mesh = plsc.VectorSubcoreMesh(
        core_axis_name="c", subcore_axis_name="s",
        num_cores=NUM_CORES, num_subcores=NUM_SUBCORES,
    )
    out = pl.kernel(
        _sc_body,
        out_type=jax.ShapeDtypeStruct((2, BATCH), jnp.float32),
        mesh=mesh,
        compiler_params=pltpu.CompilerParams(
            needs_layout_passes=False, use_tc_tiling_on_sc=False,
        ),
        scratch_types=[
            pltpu.VMEM((B_PER_W,), jnp.int32),
            pltpu.VMEM((B_PER_W,), jnp.int32),
            pltpu.VMEM((B_PER_W,), jnp.int32),
            pltpu.VMEM((TABLE_WORDS,), jnp.float32),
            pltpu.VMEM((B_PER_W,), jnp.float32),
            pltpu.VMEM((B_PER_W,), jnp.float32),
        ],
    )(stimulus_set.T, embedding.reshape(-1))
    return out.T


# speedup vs baseline: 12.0373x; 1.1794x over previous
import jax
import jax.numpy as jnp
from jax import lax
from jax.experimental import pallas as pl
from jax.experimental.pallas import tpu as pltpu
from jax.experimental.pallas import tpu_sc as plsc

N_STIMULI = 20
N_DIM = 3
BETA = 10.0
GAMMA = 0.001
BATCH = 16384

NUM_CORES = 2
NUM_SUBCORES = 16
LANES = 16
NUM_WORKERS = NUM_CORES * NUM_SUBCORES
B_PER_W = BATCH // NUM_WORKERS
STEPS = B_PER_W // LANES
TABLE_WORDS = (N_STIMULI + 1) * N_DIM
NV = N_STIMULI + 1
NPAIR = NV * NV
NPAIR_PAD = 448
NCHUNK = NPAIR_PAD // LANES


def _sqrt16(x):
    i = plsc.bitcast(x, jnp.int32)
    i = jnp.int32(0x5F3759DF) - lax.shift_right_arithmetic(i, 1)
    y = plsc.bitcast(i, jnp.float32)
    xh = x * jnp.float32(0.5)
    for _ in range(3):
        y = y * (jnp.float32(1.5) - xh * y * y)
    return x * y


def _pair_similarity(emb_v, p):
    q = p // jnp.int32(NV)
    r = p - q * jnp.int32(NV)
    q3 = q * jnp.int32(N_DIM)
    r3 = r * jnp.int32(N_DIM)
    dsq = jnp.full((LANES,), 0.0, jnp.float32)
    for d in range(N_DIM):
        dd = jnp.full((LANES,), d, jnp.int32)
        t = plsc.load_gather(emb_v, [q3 + dd]) - plsc.load_gather(emb_v, [r3 + dd])
        dsq = dsq + t * t
    return jnp.exp(jnp.float32(-BETA) * _sqrt16(dsq)) + jnp.float32(GAMMA)


def _sc_body(stim_hbm, emb_hbm, out_hbm,
             q_v, r1_v, r2_v, emb_v, stab_v, sbuf_v, p1_v, p2_v, spmem):
    sid = lax.axis_index("s")
    wid = sid * NUM_CORES + lax.axis_index("c")
    base = wid * B_PER_W

    pltpu.sync_copy(stim_hbm.at[0, pl.ds(base, B_PER_W)], q_v)
    pltpu.sync_copy(stim_hbm.at[1, pl.ds(base, B_PER_W)], r1_v)
    pltpu.sync_copy(stim_hbm.at[2, pl.ds(base, B_PER_W)], r2_v)
    pltpu.sync_copy(emb_hbm, emb_v)

    lanes = lax.iota(jnp.int32, LANES)

    p0 = jnp.minimum(sid * LANES + lanes, jnp.int32(NPAIR - 1))
    sbuf_v[...] = _pair_similarity(emb_v, p0)
    pltpu.sync_copy(sbuf_v, spmem.at[pl.ds(sid * LANES, LANES)])

    @pl.when(sid + 16 < NCHUNK)
    def _():
        p1 = jnp.minimum((sid + 16) * LANES + lanes, jnp.int32(NPAIR - 1))
        sbuf_v[...] = _pair_similarity(emb_v, p1)
        pltpu.sync_copy(sbuf_v, spmem.at[pl.ds((sid + 16) * LANES, LANES)])

    plsc.subcore_barrier()
    pltpu.sync_copy(spmem, stab_v)

    for step in range(STEPS):
        off = step * LANES
        q21 = q_v[pl.ds(off, LANES)] * jnp.int32(NV)
        s1 = plsc.load_gather(stab_v, [q21 + r1_v[pl.ds(off, LANES)]])
        s2 = plsc.load_gather(stab_v, [q21 + r2_v[pl.ds(off, LANES)]])
        inv = jnp.float32(1.0) / (s1 + s2)
        p1_v[pl.ds(off, LANES)] = s1 * inv
        p2_v[pl.ds(off, LANES)] = s2 * inv

    pltpu.sync_copy(p1_v, out_hbm.at[0, pl.ds(base, B_PER_W)])
    pltpu.sync_copy(p2_v, out_hbm.at[1, pl.ds(base, B_PER_W)])


@jax.jit
def kernel(stimulus_set, embedding):
    mesh = plsc.VectorSubcoreMesh(
        core_axis_name="c", subcore_axis_name="s",
        num_cores=NUM_CORES, num_subcores=NUM_SUBCORES,
    )
    out = pl.kernel(
        _sc_body,
        out_type=jax.ShapeDtypeStruct((2, BATCH), jnp.float32),
        mesh=mesh,
        compiler_params=pltpu.CompilerParams(
            needs_layout_passes=False, use_tc_tiling_on_sc=False,
        ),
        scratch_types=[
            pltpu.VMEM((B_PER_W,), jnp.int32),
            pltpu.VMEM((B_PER_W,), jnp.int32),
            pltpu.VMEM((B_PER_W,), jnp.int32),
            pltpu.VMEM((TABLE_WORDS,), jnp.float32),
            pltpu.VMEM((NPAIR_PAD,), jnp.float32),
            pltpu.VMEM((LANES,), jnp.float32),
            pltpu.VMEM((B_PER_W,), jnp.float32),
            pltpu.VMEM((B_PER_W,), jnp.float32),
            pltpu.VMEM_SHARED((NPAIR_PAD,), jnp.float32),
        ],
    )(stimulus_set.T, embedding.reshape(-1))
    return out.T
